# baseline (speedup 1.0000x reference)
import functools

import jax
import jax.numpy as jnp
from jax import lax
from jax.experimental import pallas as pl
from jax.experimental.pallas import tpu as pltpu
from jax.experimental.pallas import tpu_sc as plsc

_D = 64
_V = 33
_NC = 2
_NS = 16
_NW = _NC * _NS
_CHUNK = 128
_NBUF = 4


def _make_sc_call(S: int, P: int):
    s_per_w = S // _NW
    chunks_per_s = P // _CHUNK
    n_chunks = s_per_w * chunks_per_s
    n_outer = n_chunks // _NBUF
    mesh = plsc.VectorSubcoreMesh(core_axis_name="c", subcore_axis_name="s")

    @functools.partial(
        pl.kernel,
        out_type=jax.ShapeDtypeStruct((S, P, _D), jnp.float32),
        mesh=mesh,
        compiler_params=pltpu.CompilerParams(needs_layout_passes=False),
        scratch_types=(
            [pltpu.VMEM((_V, _D), jnp.float32)]
            + [pltpu.VMEM((_CHUNK, _D), jnp.float32) for _ in range(_NBUF)]
            + [pltpu.VMEM((_CHUNK,), jnp.int32) for _ in range(_NBUF)]
            + [pltpu.SemaphoreType.DMA for _ in range(3 * _NBUF)]
        ),
    )
    def sc_kernel(logits_hbm, seq_hbm, mask_hbm, out_hbm, mask_v, *rest):
        bufs = rest[:_NBUF]
        idxs = rest[_NBUF:2 * _NBUF]
        sems = rest[2 * _NBUF:]
        sem_l = sems[:_NBUF]
        sem_i = sems[_NBUF:2 * _NBUF]
        sem_o = sems[2 * _NBUF:]

        wid = lax.axis_index("s") * _NC + lax.axis_index("c")
        base_s = wid * s_per_w
        pltpu.sync_copy(mask_hbm, mask_v)

        def chunk_slice(c):
            return base_s + c // chunks_per_s, (c % chunks_per_s) * _CHUNK

        def in_copies(c, b):
            s, p0 = chunk_slice(c)
            lcp = pltpu.make_async_copy(
                logits_hbm.at[s, pl.ds(p0, _CHUNK)], bufs[b], sem_l[b])
            icp = pltpu.make_async_copy(
                seq_hbm.at[s, pl.ds(p0, _CHUNK)], idxs[b], sem_i[b])
            return lcp, icp

        def out_copy(c, b):
            s, p0 = chunk_slice(c)
            return pltpu.make_async_copy(
                bufs[b], out_hbm.at[s, pl.ds(p0, _CHUNK)], sem_o[b])

        def start_in(c, b):
            lcp, icp = in_copies(c, b)
            lcp.start()
            icp.start()

        def wait_in(c, b):
            lcp, icp = in_copies(c, b)
            lcp.wait()
            icp.wait()

        def compute(b):
            def group_body(g, carry):
                t16 = idxs[b][pl.ds(g * 16, 16)]
                offs = [t16[j] for j in range(16)]
                nq = _D // 16

                def batch_loads(j0):
                    return [mask_v[offs[j0 + r], pl.ds(q * 16, 16)]
                            for r in range(4) for q in range(nq)]

                def batch_stores(j0, ms):
                    for r in range(4):
                        row = g * 16 + j0 + r
                        for q in range(nq):
                            plsc.addupdate(
                                bufs[b].at[row, pl.ds(q * 16, 16)],
                                ms[r * nq + q])

                prev = batch_loads(0)
                for j0 in range(4, 16, 4):
                    cur = batch_loads(j0)
                    batch_stores(j0 - 4, prev)
                    prev = cur
                batch_stores(12, prev)
                return carry

            lax.fori_loop(0, _CHUNK // 16, group_body, 0, unroll=False)

        start_in(0, 0)
        start_in(1, 1)

        def outer(k, carry):
            for b in range(_NBUF):
                c = k * _NBUF + b
                wait_in(c, b)
                compute(b)
                out_copy(c, b).start()
                b2 = (b + 2) % _NBUF
                if b < 2:
                    @pl.when(k > 0)
                    def _():
                        out_copy(c - 2, b2).wait()
                    start_in(c + 2, b2)
                else:
                    out_copy(c - 2, b2).wait()

                    @pl.when(k < n_outer - 1)
                    def _():
                        start_in(c + 2, b2)
            return carry

        lax.fori_loop(0, n_outer, outer, 0, unroll=False)
        out_copy(n_chunks - 2, (n_chunks - 2) % _NBUF).wait()
        out_copy(n_chunks - 1, (n_chunks - 1) % _NBUF).wait()

    return sc_kernel


@jax.jit
def kernel(logits_SPT, seq_SP, valid_output_mask_TiTo):
    S, P, T = logits_SPT.shape
    seq = seq_SP.astype(jnp.int32)
    mask = valid_output_mask_TiTo.astype(jnp.float32)
    return _make_sc_call(S, P)(logits_SPT, seq, mask)


# speedup vs baseline: 1.3620x; 1.3620x over previous
import functools

import jax
import jax.numpy as jnp
from jax import lax
from jax.experimental import pallas as pl
from jax.experimental.pallas import tpu as pltpu
from jax.experimental.pallas import tpu_sc as plsc

_D = 64
_V = 33
_NC = 2
_NS = 16
_NW = _NC * _NS
_CHUNK = 128
_NBUF = 4


def _make_sc_call(n_rows: int):
    rows_per_w = n_rows // _NW
    n_chunks = rows_per_w // _CHUNK
    n_outer = n_chunks // _NBUF
    mesh = plsc.VectorSubcoreMesh(core_axis_name="c", subcore_axis_name="s")

    @functools.partial(
        pl.kernel,
        out_type=jax.ShapeDtypeStruct((n_rows, _D), jnp.float32),
        mesh=mesh,
        compiler_params=pltpu.CompilerParams(needs_layout_passes=False),
        scratch_types=(
            [pltpu.VMEM((_V, _D), jnp.float32)]
            + [pltpu.VMEM((_CHUNK, _D), jnp.float32) for _ in range(_NBUF)]
            + [pltpu.VMEM((_CHUNK,), jnp.int32) for _ in range(_NBUF)]
            + [pltpu.SemaphoreType.DMA for _ in range(3 * _NBUF)]
        ),
    )
    def sc_kernel(logits_hbm, seq_hbm, mask_hbm, out_hbm, mask_v, *rest):
        bufs = rest[:_NBUF]
        idxs = rest[_NBUF:2 * _NBUF]
        sems = rest[2 * _NBUF:]
        sem_l = sems[:_NBUF]
        sem_i = sems[_NBUF:2 * _NBUF]
        sem_o = sems[2 * _NBUF:]

        wid = lax.axis_index("s") * _NC + lax.axis_index("c")
        base = wid * rows_per_w
        pltpu.sync_copy(mask_hbm, mask_v)

        def in_copies(c, b):
            row0 = base + c * _CHUNK
            lcp = pltpu.make_async_copy(
                logits_hbm.at[pl.ds(row0, _CHUNK)], bufs[b], sem_l[b])
            icp = pltpu.make_async_copy(
                seq_hbm.at[pl.ds(row0, _CHUNK)], idxs[b], sem_i[b])
            return lcp, icp

        def out_copy(c, b):
            row0 = base + c * _CHUNK
            return pltpu.make_async_copy(
                bufs[b], out_hbm.at[pl.ds(row0, _CHUNK)], sem_o[b])

        def start_in(c, b):
            lcp, icp = in_copies(c, b)
            lcp.start()
            icp.start()

        def wait_in(c, b):
            lcp, icp = in_copies(c, b)
            lcp.wait()
            icp.wait()

        def compute(b):
            def group_body(g, carry):
                t16 = idxs[b][pl.ds(g * 16, 16)]
                offs = [t16[j] for j in range(16)]
                nq = _D // 16

                def batch_loads(j0):
                    return [mask_v[offs[j0 + r], pl.ds(q * 16, 16)]
                            for r in range(4) for q in range(nq)]

                def batch_stores(j0, ms):
                    for r in range(4):
                        row = g * 16 + j0 + r
                        for q in range(nq):
                            plsc.addupdate(
                                bufs[b].at[row, pl.ds(q * 16, 16)],
                                ms[r * nq + q])

                prev = batch_loads(0)
                for j0 in range(4, 16, 4):
                    cur = batch_loads(j0)
                    batch_stores(j0 - 4, prev)
                    prev = cur
                batch_stores(12, prev)
                return carry

            lax.fori_loop(0, _CHUNK // 16, group_body, 0, unroll=False)

        start_in(0, 0)
        start_in(1, 1)

        def outer(k, carry):
            for b in range(_NBUF):
                c = k * _NBUF + b
                wait_in(c, b)
                compute(b)
                out_copy(c, b).start()
                b2 = (b + 2) % _NBUF
                if b < 2:
                    @pl.when(k > 0)
                    def _():
                        out_copy(c - 2, b2).wait()
                    start_in(c + 2, b2)
                else:
                    out_copy(c - 2, b2).wait()

                    @pl.when(k < n_outer - 1)
                    def _():
                        start_in(c + 2, b2)
            return carry

        lax.fori_loop(0, n_outer, outer, 0, unroll=False)
        out_copy(n_chunks - 2, (n_chunks - 2) % _NBUF).wait()
        out_copy(n_chunks - 1, (n_chunks - 1) % _NBUF).wait()

    return sc_kernel


@jax.jit
def kernel(logits_SPT, seq_SP, valid_output_mask_TiTo):
    S, P, T = logits_SPT.shape
    n_rows = S * P
    logits2d = logits_SPT.reshape(n_rows, T)
    seq1d = seq_SP.reshape(n_rows).astype(jnp.int32)
    mask = valid_output_mask_TiTo.astype(jnp.float32)
    out = _make_sc_call(n_rows)(logits2d, seq1d, mask)
    return out.reshape(S, P, T)
